# baseline (speedup 1.0000x reference)
import functools

import jax
import jax.numpy as jnp
from jax import lax
from jax.experimental import pallas as pl
from jax.experimental.pallas import tpu as pltpu
from jax.experimental.pallas import tpu_sc as plsc

NC = 2
NS = 16
LANES = 16
NW = NC * NS


def _make_sc_kernel(batch, maxlen, dim, tchunk):
    bw = batch // NW
    n_chunks = maxlen // tchunk
    groups = bw // LANES

    mesh = plsc.VectorSubcoreMesh(core_axis_name="c", subcore_axis_name="s")

    @functools.partial(
        pl.kernel,
        out_type=jax.ShapeDtypeStruct((maxlen, dim, batch), jnp.float32),
        mesh=mesh,
        scratch_types=[
            pltpu.VMEM((tchunk, bw), jnp.int32),
            pltpu.VMEM((tchunk, bw, dim), jnp.float32),
            pltpu.VMEM((tchunk, dim, bw), jnp.float32),
            pltpu.VMEM((tchunk, dim, LANES), jnp.float32),
            pltpu.SemaphoreType.DMA,
        ],
        compiler_params=pltpu.CompilerParams(
            use_tc_tiling_on_sc=False, needs_layout_passes=False
        ),
    )
    def kern(x4_hbm, tok_hbm, pos_hbm, out_hbm, idx_v, rows_v, blk_v, pos_v, sem):
        wid = lax.axis_index("s") * NC + lax.axis_index("c")
        b0 = wid * bw

        lanes = lax.iota(jnp.int32, LANES)
        rowq = [lanes + (q * LANES) for q in range(groups)]

        def do_chunk(g, carry):
            t0 = g * tchunk
            pltpu.sync_copy(pos_hbm.at[pl.ds(t0, tchunk)], pos_v)
            pltpu.sync_copy(x4_hbm.at[pl.ds(t0, tchunk), pl.ds(b0, bw)], idx_v)
            for j in range(tchunk):
                pltpu.async_copy(tok_hbm.at[idx_v.at[j]], rows_v.at[j], sem)
            for j in range(tchunk):
                pltpu.make_async_copy(
                    tok_hbm.at[idx_v.at[j]], rows_v.at[j], sem
                ).wait()

            @plsc.parallel_loop(0, tchunk, 1, unroll=2)
            def transpose_add(j):
                src = rows_v.at[j]
                dst = blk_v.at[j]
                pos_j = pos_v.at[j]
                for f in range(dim):
                    pv = pos_j[f, pl.ds(0, LANES)]
                    ff = jnp.full((LANES,), f, jnp.int32)
                    for q in range(groups):
                        vals = plsc.load_gather(src, [rowq[q], ff])
                        dst[f, pl.ds(q * LANES, LANES)] = vals + pv

            pltpu.sync_copy(
                blk_v, out_hbm.at[pl.ds(t0, tchunk), :, pl.ds(b0, bw)]
            )
            return carry

        lax.fori_loop(0, n_chunks, do_chunk, None)

    return kern


@jax.jit
def kernel(x, token_emb, pos_emb):
    batch, maxlen = x.shape
    vocab, dim = token_emb.shape
    pad = 128 // dim
    tok4 = jnp.pad(token_emb, ((0, 0), (0, 128 - dim))).reshape(vocab * pad, dim)
    x4 = (x * pad).T.astype(jnp.int32)
    posb = jnp.broadcast_to(pos_emb[:, :, None], (maxlen, dim, LANES))
    k = _make_sc_kernel(batch, maxlen, dim, tchunk=8)
    out_t = k(x4, tok4, posb)
    return out_t.transpose(2, 0, 1)


# speedup vs baseline: 1.2152x; 1.2152x over previous
import functools

import jax
import jax.numpy as jnp
from jax import lax
from jax.experimental import pallas as pl
from jax.experimental.pallas import tpu as pltpu
from jax.experimental.pallas import tpu_sc as plsc

NC = 2
NS = 16
LANES = 16
NW = NC * NS


def _make_sc_kernel(batch, maxlen, dim, tchunk):
    bw = batch // NW
    n_chunks = maxlen // tchunk
    groups = bw // LANES

    mesh = plsc.VectorSubcoreMesh(core_axis_name="c", subcore_axis_name="s")

    @functools.partial(
        pl.kernel,
        out_type=jax.ShapeDtypeStruct((maxlen, dim, batch), jnp.float32),
        mesh=mesh,
        scratch_types=[
            pltpu.VMEM((tchunk, bw), jnp.int32),
            pltpu.VMEM((tchunk, bw, dim), jnp.float32),
            pltpu.VMEM((tchunk, dim, bw), jnp.float32),
            pltpu.VMEM((tchunk, dim, LANES), jnp.float32),
            pltpu.SemaphoreType.DMA,
        ],
        compiler_params=pltpu.CompilerParams(
            use_tc_tiling_on_sc=False, needs_layout_passes=False
        ),
    )
    def kern(x4_hbm, tok_hbm, pos_hbm, out_hbm, idx_v, rows_v, blk_v, pos_v, sem):
        wid = lax.axis_index("s") * NC + lax.axis_index("c")
        b0 = wid * bw

        lanes = lax.iota(jnp.int32, LANES)
        rowq = [lanes + (q * LANES) for q in range(groups)]

        def do_chunk(g, carry):
            t0 = g * tchunk
            pltpu.sync_copy(pos_hbm.at[pl.ds(t0, tchunk)], pos_v)
            pltpu.sync_copy(x4_hbm.at[pl.ds(t0, tchunk), pl.ds(b0, bw)], idx_v)
            for j in range(tchunk):
                pltpu.async_copy(tok_hbm.at[idx_v.at[j]], rows_v.at[j], sem)
            for j in range(tchunk):
                pltpu.make_async_copy(
                    tok_hbm.at[idx_v.at[j]], rows_v.at[j], sem
                ).wait()

            @plsc.parallel_loop(0, tchunk, 1)
            def transpose_add(j):
                src = rows_v.at[j]
                dst = blk_v.at[j]
                pos_j = pos_v.at[j]
                for f in range(dim):
                    pv = pos_j[f, pl.ds(0, LANES)]
                    ff = jnp.full((LANES,), f, jnp.int32)
                    vals = [
                        plsc.load_gather(src, [rowq[q], ff])
                        for q in range(groups)
                    ]
                    for q in range(groups):
                        dst[f, pl.ds(q * LANES, LANES)] = vals[q] + pv

            pltpu.sync_copy(
                blk_v, out_hbm.at[pl.ds(t0, tchunk), :, pl.ds(b0, bw)]
            )
            return carry

        lax.fori_loop(0, n_chunks, do_chunk, None)

    return kern


@jax.jit
def kernel(x, token_emb, pos_emb):
    batch, maxlen = x.shape
    vocab, dim = token_emb.shape
    pad = 128 // dim
    tok4 = jnp.pad(token_emb, ((0, 0), (0, 128 - dim))).reshape(vocab * pad, dim)
    x4 = (x * pad).T.astype(jnp.int32)
    posb = jnp.broadcast_to(pos_emb[:, :, None], (maxlen, dim, LANES))
    k = _make_sc_kernel(batch, maxlen, dim, tchunk=8)
    out_t = k(x4, tok4, posb)
    return out_t.transpose(2, 0, 1)
